# baseline (speedup 1.0000x reference)
import functools

import jax
import jax.numpy as jnp
from jax import lax
from jax.experimental import pallas as pl
from jax.experimental.pallas import tpu as pltpu
from jax.experimental.pallas import tpu_sc as plsc

HIDDEN = 128
NUM_SCALES = 100000
BATCH = 16384

NC = 2
NS = 16
LANES = 16
NW = NC * NS
BPW = BATCH // NW
CHUNK = 128
NCHUNK = BPW // CHUNK
REGS_PER_CHUNK = CHUNK // LANES


def _body(scale_hbm, table_hbm, out_hbm, idx_v, rows_v, osem, *gsems):
    wid = lax.axis_index("s") * NC + lax.axis_index("c")
    base = wid * BPW

    pltpu.sync_copy(scale_hbm.at[pl.ds(base, BPW)], idx_v)

    table2d = table_hbm.at[0]
    out2d = out_hbm.at[0]

    gathers = []
    for c in range(NCHUNK):
        for i in range(c * REGS_PER_CHUNK, (c + 1) * REGS_PER_CHUNK):
            v = idx_v[pl.ds(i * LANES, LANES)]
            idx_v[pl.ds(i * LANES, LANES)] = (
                jnp.minimum(jnp.maximum(v, 0), NUM_SCALES - 1) + 1
            )
        gathers.append(
            pltpu.async_copy(
                table2d.at[idx_v.at[pl.ds(c * CHUNK, CHUNK)]],
                rows_v.at[pl.ds(c * CHUNK, CHUNK)],
                gsems[c],
            )
        )

    outs = []
    for c in range(NCHUNK):
        gathers[c].wait()
        outs.append(
            pltpu.async_copy(
                rows_v.at[pl.ds(c * CHUNK, CHUNK)],
                out2d.at[pl.ds(base + c * CHUNK, CHUNK)],
                osem,
            )
        )
    for o in outs:
        o.wait()


@functools.cache
def _build():
    mesh = plsc.VectorSubcoreMesh(
        core_axis_name="c", subcore_axis_name="s", num_cores=NC, num_subcores=NS
    )
    return pl.kernel(
        _body,
        out_type=jax.ShapeDtypeStruct((1, BATCH, HIDDEN), jnp.float32),
        mesh=mesh,
        scratch_types=[
            pltpu.VMEM((BPW,), jnp.int32),
            pltpu.VMEM((BPW, HIDDEN), jnp.float32),
            pltpu.SemaphoreType.DMA,
        ]
        + [pltpu.SemaphoreType.DMA] * NCHUNK,
        compiler_params=pltpu.CompilerParams(use_tc_tiling_on_sc=False),
    )


def kernel(scale, scale_embeddings):
    idx = scale.reshape(BATCH).astype(jnp.int32)
    return _build()(idx, scale_embeddings)


# speedup vs baseline: 1.2122x; 1.2122x over previous
import functools

import jax
import jax.numpy as jnp
from jax import lax
from jax.experimental import pallas as pl
from jax.experimental.pallas import tpu as pltpu
from jax.experimental.pallas import tpu_sc as plsc

HIDDEN = 128
NUM_SCALES = 100000
BATCH = 16384

NC = 2
NS = 16
LANES = 16
NW = NC * NS
BPW = BATCH // NW
CHUNK = 128
NCHUNK = BPW // CHUNK
REGS_PER_CHUNK = CHUNK // LANES


def _body(scale_hbm, table_hbm, out_hbm, idx_v, rows_v, osem, *gsems):
    wid = lax.axis_index("s") * NC + lax.axis_index("c")
    base = wid * BPW

    pltpu.sync_copy(scale_hbm.at[pl.ds(base, BPW)], idx_v)

    table2d = table_hbm.at[0]
    out2d = out_hbm.at[0]

    gathers = []
    for c in range(NCHUNK):
        for i in range(c * REGS_PER_CHUNK, (c + 1) * REGS_PER_CHUNK):
            v = idx_v[pl.ds(i * LANES, LANES)]
            idx_v[pl.ds(i * LANES, LANES)] = (
                jnp.minimum(jnp.maximum(v, 0), NUM_SCALES - 1) + 1
            )
        gathers.append(
            pltpu.async_copy(
                table2d.at[idx_v.at[pl.ds(c * CHUNK, CHUNK)]],
                rows_v.at[pl.ds(c * CHUNK, CHUNK)],
                gsems[c],
            )
        )

    outs = []
    for c in range(NCHUNK):
        gathers[c].wait()
        outs.append(
            pltpu.async_copy(
                rows_v.at[pl.ds(c * CHUNK, CHUNK)],
                out2d.at[pl.ds(base + c * CHUNK, CHUNK)],
                osem,
            )
        )
    for o in outs:
        o.wait()


@functools.cache
def _build():
    mesh = plsc.VectorSubcoreMesh(
        core_axis_name="c", subcore_axis_name="s", num_cores=NC, num_subcores=NS
    )
    return pl.kernel(
        _body,
        out_type=jax.ShapeDtypeStruct((1, BATCH, HIDDEN), jnp.float32),
        mesh=mesh,
        scratch_types=[
            pltpu.VMEM((BPW,), jnp.int32),
            pltpu.VMEM((BPW, HIDDEN), jnp.float32),
            pltpu.SemaphoreType.DMA,
        ]
        + [pltpu.SemaphoreType.DMA] * NCHUNK,
        compiler_params=pltpu.CompilerParams(needs_layout_passes=False),
    )


def kernel(scale, scale_embeddings):
    idx = scale.reshape(BATCH).astype(jnp.int32)
    return _build()(idx, scale_embeddings)


# speedup vs baseline: 2.5473x; 2.1014x over previous
import functools

import jax
import jax.numpy as jnp
from jax import lax
from jax.experimental import pallas as pl
from jax.experimental.pallas import tpu as pltpu
from jax.experimental.pallas import tpu_sc as plsc

HIDDEN = 128
NUM_SCALES = 100000
BATCH = 16384

NC = 2
NS = 16
LANES = 16
NW = NC * NS
BPW = BATCH // NW
CHUNK = 128
NCHUNK = BPW // CHUNK
REGS_PER_CHUNK = CHUNK // LANES


def _body(scale_hbm, table_hbm, out_hbm, idx_v, rows_v, osem, *gsems):
    wid = lax.axis_index("s") * NC + lax.axis_index("c")
    base = wid * BPW

    pltpu.sync_copy(scale_hbm.at[pl.ds(base, BPW)], idx_v)

    table2d = table_hbm.at[:, 0]
    out2d = out_hbm.at[0]

    gathers = []
    for c in range(NCHUNK):
        for i in range(c * REGS_PER_CHUNK, (c + 1) * REGS_PER_CHUNK):
            v = idx_v[pl.ds(i * LANES, LANES)]
            idx_v[pl.ds(i * LANES, LANES)] = (
                jnp.minimum(jnp.maximum(v, 0), NUM_SCALES - 1) + 1
            )
        gathers.append(
            pltpu.async_copy(
                table2d.at[idx_v.at[pl.ds(c * CHUNK, CHUNK)]],
                rows_v.at[pl.ds(c * CHUNK, CHUNK)],
                gsems[c],
            )
        )

    outs = []
    for c in range(NCHUNK):
        gathers[c].wait()
        outs.append(
            pltpu.async_copy(
                rows_v.at[pl.ds(c * CHUNK, CHUNK)],
                out2d.at[pl.ds(base + c * CHUNK, CHUNK)],
                osem,
            )
        )
    for o in outs:
        o.wait()


@functools.cache
def _build():
    mesh = plsc.VectorSubcoreMesh(
        core_axis_name="c", subcore_axis_name="s", num_cores=NC, num_subcores=NS
    )
    return pl.kernel(
        _body,
        out_type=jax.ShapeDtypeStruct((1, BATCH, HIDDEN), jnp.float32),
        mesh=mesh,
        scratch_types=[
            pltpu.VMEM((BPW,), jnp.int32),
            pltpu.VMEM((BPW, HIDDEN), jnp.float32),
            pltpu.SemaphoreType.DMA,
        ]
        + [pltpu.SemaphoreType.DMA] * NCHUNK,
    )


def kernel(scale, scale_embeddings):
    idx = scale.reshape(BATCH).astype(jnp.int32)
    table3d = scale_embeddings.transpose(1, 0, 2)
    return _build()(idx, table3d)
